# baseline (speedup 1.0000x reference)
import numpy as np
import jax
import jax.numpy as jnp
from jax.experimental import pallas as pl

_ROWS = 1024 * 128
_C = 192
_BLOCK_ROWS = 1024


def _perm() -> np.ndarray:
    mixed = np.stack([np.arange(48, 96), np.arange(96, 144)]).T.reshape(-1)
    return np.concatenate([np.arange(0, 48), mixed, np.arange(144, 192)])


def _perm_matrix() -> np.ndarray:
    p = np.zeros((_C, _C), dtype=np.float32)
    p[_perm(), np.arange(_C)] = 1.0
    return p


def _permute_body(x_ref, p_ref, o_ref):
    o_ref[...] = jnp.dot(x_ref[...], p_ref[...],
                         preferred_element_type=jnp.float32)


def _permute_body3(x_ref, p_ref, o_ref):
    b = x_ref.shape[0] * x_ref.shape[1]
    x = x_ref[...].reshape(b, _C)
    o_ref[...] = jnp.dot(x, p_ref[...],
                         preferred_element_type=jnp.float32).reshape(x_ref.shape)


def kernel(inputs):
    n, t, _ = inputs.shape
    bn = 8
    p = jnp.asarray(_perm_matrix())
    return pl.pallas_call(
        _permute_body3,
        grid=(n // bn,),
        in_specs=[
            pl.BlockSpec((bn, t, _C), lambda i: (i, 0, 0)),
            pl.BlockSpec((_C, _C), lambda i: (0, 0)),
        ],
        out_specs=pl.BlockSpec((bn, t, _C), lambda i: (i, 0, 0)),
        out_shape=jax.ShapeDtypeStruct((n, t, _C), jnp.float32),
    )(inputs, p)


# speedup vs baseline: 2.8005x; 2.8005x over previous
import numpy as np
import jax
import jax.numpy as jnp
from jax.experimental import pallas as pl

_C = 192
_BN = 8


def _permute_body(x_ref, o_ref):
    x = x_ref[...]
    a = x[:, 48:96, :]
    b = x[:, 96:144, :]
    mid = jnp.stack([a, b], axis=2).reshape(x.shape[0], 96, x.shape[2])
    o_ref[...] = jnp.concatenate([x[:, :48, :], mid, x[:, 144:, :]], axis=1)


def kernel(inputs):
    n, t, _ = inputs.shape
    xt = jnp.swapaxes(inputs, 1, 2)
    out = pl.pallas_call(
        _permute_body,
        grid=(n // _BN,),
        in_specs=[pl.BlockSpec((_BN, _C, t), lambda i: (i, 0, 0))],
        out_specs=pl.BlockSpec((_BN, _C, t), lambda i: (i, 0, 0)),
        out_shape=jax.ShapeDtypeStruct((n, _C, t), jnp.float32),
    )(xt)
    return jnp.swapaxes(out, 1, 2)


# speedup vs baseline: 3.1724x; 1.1328x over previous
import numpy as np
import jax
import jax.numpy as jnp
from jax.experimental import pallas as pl

_C = 192
_BN = 8


def _permute_body(x_ref, o_ref):
    o_ref[:, 0:48, :] = x_ref[:, 0:48, :]
    o_ref[:, 48:144:2, :] = x_ref[:, 48:96, :]
    o_ref[:, 49:144:2, :] = x_ref[:, 96:144, :]
    o_ref[:, 144:192, :] = x_ref[:, 144:192, :]


def kernel(inputs):
    n, t, _ = inputs.shape
    xt = jnp.swapaxes(inputs, 1, 2)
    out = pl.pallas_call(
        _permute_body,
        grid=(n // _BN,),
        in_specs=[pl.BlockSpec((_BN, _C, t), lambda i: (i, 0, 0))],
        out_specs=pl.BlockSpec((_BN, _C, t), lambda i: (i, 0, 0)),
        out_shape=jax.ShapeDtypeStruct((n, _C, t), jnp.float32),
    )(xt)
    return jnp.swapaxes(out, 1, 2)


# speedup vs baseline: 5.4208x; 1.7087x over previous
import numpy as np
import jax
import jax.numpy as jnp
from jax.experimental import pallas as pl

_C = 192
_BN = 32


def _permute_body(x_ref, o_ref):
    o_ref[:, 0:48, :] = x_ref[:, 0:48, :]
    o_ref[:, 48:144:2, :] = x_ref[:, 48:96, :]
    o_ref[:, 49:144:2, :] = x_ref[:, 96:144, :]
    o_ref[:, 144:192, :] = x_ref[:, 144:192, :]


def kernel(inputs):
    n, t, _ = inputs.shape
    xt = jnp.swapaxes(inputs, 1, 2)
    out = pl.pallas_call(
        _permute_body,
        grid=(n // _BN,),
        in_specs=[pl.BlockSpec((_BN, _C, t), lambda i: (i, 0, 0))],
        out_specs=pl.BlockSpec((_BN, _C, t), lambda i: (i, 0, 0)),
        out_shape=jax.ShapeDtypeStruct((n, _C, t), jnp.float32),
    )(xt)
    return jnp.swapaxes(out, 1, 2)


# speedup vs baseline: 5.6562x; 1.0434x over previous
import numpy as np
import jax
import jax.numpy as jnp
from jax.experimental import pallas as pl

_C = 192
_BN = 64


def _permute_body(x_ref, o_ref):
    o_ref[:, 0:48, :] = x_ref[:, 0:48, :]
    o_ref[:, 48:144:2, :] = x_ref[:, 48:96, :]
    o_ref[:, 49:144:2, :] = x_ref[:, 96:144, :]
    o_ref[:, 144:192, :] = x_ref[:, 144:192, :]


def kernel(inputs):
    n, t, _ = inputs.shape
    xt = jnp.swapaxes(inputs, 1, 2)
    out = pl.pallas_call(
        _permute_body,
        grid=(n // _BN,),
        in_specs=[pl.BlockSpec((_BN, _C, t), lambda i: (i, 0, 0))],
        out_specs=pl.BlockSpec((_BN, _C, t), lambda i: (i, 0, 0)),
        out_shape=jax.ShapeDtypeStruct((n, _C, t), jnp.float32),
    )(xt)
    return jnp.swapaxes(out, 1, 2)


# speedup vs baseline: 5.6898x; 1.0059x over previous
import numpy as np
import jax
import jax.numpy as jnp
from jax.experimental import pallas as pl

_C = 192
_BN = 128


def _permute_body(x_ref, o_ref):
    o_ref[:, 0:48, :] = x_ref[:, 0:48, :]
    o_ref[:, 48:144:2, :] = x_ref[:, 48:96, :]
    o_ref[:, 49:144:2, :] = x_ref[:, 96:144, :]
    o_ref[:, 144:192, :] = x_ref[:, 144:192, :]


def kernel(inputs):
    n, t, _ = inputs.shape
    xt = jnp.swapaxes(inputs, 1, 2)
    out = pl.pallas_call(
        _permute_body,
        grid=(n // _BN,),
        in_specs=[pl.BlockSpec((_BN, _C, t), lambda i: (i, 0, 0))],
        out_specs=pl.BlockSpec((_BN, _C, t), lambda i: (i, 0, 0)),
        out_shape=jax.ShapeDtypeStruct((n, _C, t), jnp.float32),
    )(xt)
    return jnp.swapaxes(out, 1, 2)
